# Initial kernel scaffold; baseline (speedup 1.0000x reference)
import jax
import jax.numpy as jnp
from jax.experimental import pallas as pl


def kernel(input_ids, attention_mask, embed, final_norm, lm_head_T, ln1, wqkvT, woT, ln2, wguT, wdT, cos_qk, sin_qk, rot_qk):
    raise NotImplementedError("write your pallas kernel here")


# speedup vs baseline: 14.5864x; 14.5864x over previous
import math

import jax
import jax.numpy as jnp
from jax import lax
from jax.experimental import pallas as pl
from jax.experimental.pallas import tpu as pltpu

SEQ = 8
HIDDEN = 64
N_HEADS = 4
HEAD_DIM = HIDDEN // N_HEADS
INTER = 128
VOCAB = 256
N_LAYERS = 2
EPS = 1e-6
NB = 16
ROWS = NB * SEQ


def _fwd_kernel(ids_ref, mask_ref, embed_ref, ln1_ref, wqkv_ref, wo_ref,
                ln2_ref, wgu_ref, wd_ref, cos_ref, sin_ref, rot_ref,
                fnorm_ref, wlm_ref, out_ref):
    f32 = jnp.float32

    ids = ids_ref[...]
    vocab_iota = lax.broadcasted_iota(jnp.int32, (ROWS, VOCAB), 1)
    onehot = (ids == vocab_iota).astype(f32)
    x = jnp.dot(onehot, embed_ref[...], preferred_element_type=f32)

    row = lax.broadcasted_iota(jnp.int32, (ROWS, ROWS), 0)
    col = lax.broadcasted_iota(jnp.int32, (ROWS, ROWS), 1)
    same = (row // SEQ) == (col // SEQ)
    keep = jnp.logical_and(jnp.logical_and(same, col <= row),
                           mask_ref[0] > 0.5)
    bias = jnp.where(keep, f32(0.0),
                     jnp.where(same, f32(-1e9), f32(-2e9)))

    head_id = lax.broadcasted_iota(jnp.int32, (1, HIDDEN), 1) // HEAD_DIM
    scale = f32(1.0 / math.sqrt(HEAD_DIM))

    for l in range(N_LAYERS):
        var = jnp.mean(x * x, axis=-1, keepdims=True)
        xn = x * lax.rsqrt(var + EPS) * ln1_ref[l]

        qkv = jnp.dot(xn, wqkv_ref[l], preferred_element_type=f32)
        qk = qkv[:, :2 * HIDDEN]
        v = qkv[:, 2 * HIDDEN:]

        qk = qk * cos_ref[...] + jnp.dot(qk, rot_ref[...],
                                         preferred_element_type=f32) * sin_ref[...]
        q = qk[:, :HIDDEN]
        k = qk[:, HIDDEN:]

        attn = jnp.zeros((ROWS, HIDDEN), f32)
        for h in range(N_HEADS):
            mh = (head_id == h).astype(f32)
            km = k * mh
            vm = v * mh
            s = lax.dot_general(q, km, (((1,), (1,)), ((), ())),
                                preferred_element_type=f32) * scale + bias
            s = s - jnp.max(s, axis=-1, keepdims=True)
            p = jnp.exp(s)
            p = p * pl.reciprocal(jnp.sum(p, axis=-1, keepdims=True),
                                  approx=True)
            attn = attn + jnp.dot(p, vm, preferred_element_type=f32)

        x = x + jnp.dot(attn, wo_ref[l], preferred_element_type=f32)

        var2 = jnp.mean(x * x, axis=-1, keepdims=True)
        xn2 = x * lax.rsqrt(var2 + EPS) * ln2_ref[l]
        gu = jnp.dot(xn2, wgu_ref[l], preferred_element_type=f32)
        g = gu[:, :INTER]
        u = gu[:, INTER:]
        x = x + jnp.dot(g * jax.nn.sigmoid(g) * u, wd_ref[l],
                        preferred_element_type=f32)

    varf = jnp.mean(x * x, axis=-1, keepdims=True)
    xf = x * lax.rsqrt(varf + EPS) * fnorm_ref[...]
    out_ref[...] = jnp.dot(xf, wlm_ref[...], preferred_element_type=f32)


def kernel(input_ids, attention_mask, embed, final_norm, lm_head_T,
           ln1, wqkvT, woT, ln2, wguT, wdT, cos_qk, sin_qk, rot_qk):
    b = input_ids.shape[0]
    nblk = b // NB

    ids_flat = input_ids.reshape(b * SEQ, 1)
    mask_rows = attention_mask.reshape(nblk, 1, ROWS)
    cos_full = jnp.tile(cos_qk, (NB, 1))
    sin_full = jnp.tile(sin_qk, (NB, 1))

    shared = lambda shape: pl.BlockSpec(shape, lambda g: tuple(0 for _ in shape))

    out = pl.pallas_call(
        _fwd_kernel,
        out_shape=jax.ShapeDtypeStruct((b * SEQ, VOCAB), jnp.float32),
        grid=(nblk,),
        in_specs=[
            pl.BlockSpec((ROWS, 1), lambda g: (g, 0)),
            pl.BlockSpec((1, 1, ROWS), lambda g: (g, 0, 0)),
            shared((VOCAB, HIDDEN)),
            shared((N_LAYERS, 1, HIDDEN)),
            shared((N_LAYERS, HIDDEN, 3 * HIDDEN)),
            shared((N_LAYERS, HIDDEN, HIDDEN)),
            shared((N_LAYERS, 1, HIDDEN)),
            shared((N_LAYERS, HIDDEN, 2 * INTER)),
            shared((N_LAYERS, INTER, HIDDEN)),
            shared((ROWS, 2 * HIDDEN)),
            shared((ROWS, 2 * HIDDEN)),
            shared((2 * HIDDEN, 2 * HIDDEN)),
            shared((1, HIDDEN)),
            shared((HIDDEN, VOCAB)),
        ],
        out_specs=pl.BlockSpec((ROWS, VOCAB), lambda g: (g, 0)),
        compiler_params=pltpu.CompilerParams(
            dimension_semantics=("parallel",)),
    )(ids_flat, mask_rows, embed, ln1, wqkvT, woT, ln2, wguT, wdT,
      cos_full, sin_full, rot_qk, final_norm, lm_head_T)

    return out.reshape(b, SEQ, VOCAB)


# speedup vs baseline: 46.3761x; 3.1794x over previous
import jax
import jax.numpy as jnp
from jax import lax
from jax.experimental import pallas as pl
from jax.experimental.pallas import tpu as pltpu

SEQ = 8
HIDDEN = 64
N_HEADS = 4
HEAD_DIM = HIDDEN // N_HEADS
INTER = 128
VOCAB = 256
N_LAYERS = 2
EPS = 1e-6
ROWS = 512
TILE = 128
N_TILES = ROWS // TILE
BF = jnp.bfloat16
F32 = jnp.float32


def _fwd_kernel(ids_ref, mask_ref, embed_ref, wqkv_ref, wo_ref,
                wgu_ref, wd_ref, cos_ref, sin_ref, rot_ref,
                wlm_ref, jmean_ref, out_ref):
    ids = ids_ref[...]
    vocab_iota = lax.broadcasted_iota(jnp.int32, (ROWS, VOCAB), 1)
    onehot = (ids == vocab_iota).astype(BF)
    x = jnp.dot(onehot, embed_ref[...], preferred_element_type=F32)

    row = lax.broadcasted_iota(jnp.int32, (TILE, TILE), 0)
    col = lax.broadcasted_iota(jnp.int32, (TILE, TILE), 1)
    base = jnp.logical_and((row // SEQ) == (col // SEQ), col <= row)
    samef = ((row // SEQ) == (col // SEQ)).astype(F32)
    mrow = mask_ref[0]
    pmasks, alives = [], []
    for t in range(N_TILES):
        mk = mrow[:, t * TILE:(t + 1) * TILE] > 0.5
        keepf = jnp.logical_and(base, mk).astype(F32)
        alive = jnp.max(keepf, axis=-1, keepdims=True)
        pmask = jnp.maximum(keepf, samef * (1.0 - alive))
        pmasks.append(pmask.astype(BF))
        alives.append(alive)

    ones_sum = jnp.ones((TILE, HIDDEN), BF)

    for l in range(N_LAYERS):
        mean = jnp.dot(x * x, jmean_ref[...], preferred_element_type=F32)
        xn = x * lax.rsqrt(mean + EPS)

        qkv = jnp.dot(xn.astype(BF), wqkv_ref[l],
                      preferred_element_type=F32)
        qk = qkv[:, :2 * HIDDEN]
        v = qkv[:, 2 * HIDDEN:3 * HIDDEN]
        qk = qk * cos_ref[...] + jnp.dot(qk.astype(BF), rot_ref[...],
                                         preferred_element_type=F32) * sin_ref[...]
        qb = qk[:, :HIDDEN].astype(BF)
        kb = qk[:, HIDDEN:].astype(BF)
        vb = v.astype(BF)

        head_id = lax.broadcasted_iota(jnp.int32, (1, HIDDEN), 1) // HEAD_DIM
        attn_tiles = [None] * N_TILES
        for h in range(N_HEADS):
            mh = (head_id == h).astype(BF)
            km = kb * mh
            vm = vb * mh
            for t in range(N_TILES):
                sl = slice(t * TILE, (t + 1) * TILE)
                qa = (qb[sl] * alives[t].astype(BF))
                s = lax.dot_general(qa, km[sl], (((1,), (1,)), ((), ())),
                                    preferred_element_type=F32)
                p = (jnp.exp(s) * pmasks[t]).astype(BF)
                pv = jnp.dot(p, vm[sl], preferred_element_type=F32)
                den = jnp.dot(p, ones_sum, preferred_element_type=F32)
                a = pv * pl.reciprocal(den, approx=True)
                attn_tiles[t] = a if attn_tiles[t] is None else attn_tiles[t] + a
        attn = jnp.concatenate(attn_tiles, axis=0)

        x = x + jnp.dot(attn.astype(BF), wo_ref[l], preferred_element_type=F32)

        mean2 = jnp.dot(x * x, jmean_ref[...], preferred_element_type=F32)
        xn2 = x * lax.rsqrt(mean2 + EPS)
        gu = jnp.dot(xn2.astype(BF), wgu_ref[l], preferred_element_type=F32)
        g = gu[:, :INTER]
        u = gu[:, INTER:]
        hid = g * jax.nn.sigmoid(g) * u
        x = x + jnp.dot(hid.astype(BF), wd_ref[l], preferred_element_type=F32)

    meanf = jnp.dot(x * x, jmean_ref[...], preferred_element_type=F32)
    xf = x * lax.rsqrt(meanf + EPS)
    out_ref[...] = jnp.dot(xf.astype(BF), wlm_ref[...],
                           preferred_element_type=F32)


def kernel(input_ids, attention_mask, embed, final_norm, lm_head_T,
           ln1, wqkvT, woT, ln2, wguT, wdT, cos_qk, sin_qk, rot_qk):
    b = input_ids.shape[0]
    nblk = (b * SEQ) // ROWS
    nbseq = ROWS // SEQ

    scale = 1.0 / (HEAD_DIM ** 0.5)
    ln1c = jnp.swapaxes(ln1, 1, 2)
    ln2c = jnp.swapaxes(ln2, 1, 2)
    wq = wqkvT[:, :, :HIDDEN] * (ln1c * scale)
    wk = wqkvT[:, :, HIDDEN:2 * HIDDEN] * ln1c
    wv = wqkvT[:, :, 2 * HIDDEN:] * ln1c
    pad = jnp.zeros((N_LAYERS, HIDDEN, HIDDEN), F32)
    wqkv_p = jnp.concatenate([wq, wk, wv, pad], axis=-1).astype(BF)
    wgu_f = (wguT * ln2c).astype(BF)
    wlm_f = (lm_head_T * jnp.swapaxes(final_norm, 0, 1)).astype(BF)
    wo_b = woT.astype(BF)
    wd_b = wdT.astype(BF)
    rot_b = rot_qk.astype(BF)
    jmean = jnp.full((HIDDEN, HIDDEN), 1.0 / HIDDEN, F32)

    ids_flat = input_ids.reshape(b * SEQ, 1)
    mask_rows = attention_mask.reshape(nblk, 1, ROWS)
    cos_full = jnp.tile(cos_qk, (nbseq, 1))
    sin_full = jnp.tile(sin_qk, (nbseq, 1))

    shared = lambda shape: pl.BlockSpec(shape, lambda g: tuple(0 for _ in shape))

    out = pl.pallas_call(
        _fwd_kernel,
        out_shape=jax.ShapeDtypeStruct((b * SEQ, VOCAB), F32),
        grid=(nblk,),
        in_specs=[
            pl.BlockSpec((ROWS, 1), lambda g: (g, 0)),
            pl.BlockSpec((1, 1, ROWS), lambda g: (g, 0, 0)),
            shared((VOCAB, HIDDEN)),
            shared((N_LAYERS, HIDDEN, 4 * HIDDEN)),
            shared((N_LAYERS, HIDDEN, HIDDEN)),
            shared((N_LAYERS, HIDDEN, 2 * INTER)),
            shared((N_LAYERS, INTER, HIDDEN)),
            shared((ROWS, 2 * HIDDEN)),
            shared((ROWS, 2 * HIDDEN)),
            shared((2 * HIDDEN, 2 * HIDDEN)),
            shared((HIDDEN, VOCAB)),
            shared((HIDDEN, HIDDEN)),
        ],
        out_specs=pl.BlockSpec((ROWS, VOCAB), lambda g: (g, 0)),
        compiler_params=pltpu.CompilerParams(
            dimension_semantics=("parallel",)),
    )(ids_flat, mask_rows, embed, wqkv_p, wo_b, wgu_f, wd_b,
      cos_full, sin_full, rot_b, wlm_f, jmean)

    return out.reshape(b, SEQ, VOCAB)


# speedup vs baseline: 59.8053x; 1.2896x over previous
import jax
import jax.numpy as jnp
from jax import lax
from jax.experimental import pallas as pl
from jax.experimental.pallas import tpu as pltpu

SEQ = 8
HIDDEN = 64
N_HEADS = 4
HEAD_DIM = HIDDEN // N_HEADS
INTER = 128
VOCAB = 256
N_LAYERS = 2
EPS = 1e-6
ROWS = 512
TILE = 128
N_TILES = ROWS // TILE
BF = jnp.bfloat16
F32 = jnp.float32


def _fwd_kernel(ids_ref, mask_ref, embed_ref, wqkv_ref, wo_ref,
                wgu_ref, wd_ref, cos_ref, sin_ref, rot_ref,
                wlm_ref, out_ref):
    dn = lambda: (((1,), (0,)), ((), ()))
    dt = lambda: (((0,), (0,)), ((), ()))

    ids = ids_ref[0]
    vocab_iota = lax.broadcasted_iota(jnp.int32, (VOCAB, ROWS), 0)
    onehot = (vocab_iota == ids).astype(BF)
    xt = lax.dot_general(embed_ref[...], onehot, dn(),
                         preferred_element_type=F32)

    krow = lax.broadcasted_iota(jnp.int32, (TILE, TILE), 0)
    qcol = lax.broadcasted_iota(jnp.int32, (TILE, TILE), 1)
    samef = ((krow // SEQ) == (qcol // SEQ)).astype(F32)
    base = jnp.logical_and((krow // SEQ) == (qcol // SEQ), krow <= qcol)
    mkey = mask_ref[0]
    pmask4s, alives = [], []
    for t in range(N_TILES):
        mk = mkey[t * TILE:(t + 1) * TILE] > 0.5
        keepf = jnp.logical_and(base, mk).astype(F32)
        alive = jnp.max(keepf, axis=0, keepdims=True)
        pmask = jnp.maximum(keepf, samef * (1.0 - alive))
        pmask4s.append(jnp.concatenate([pmask] * N_HEADS, axis=0).astype(BF))
        alives.append(alive.astype(BF))

    hm = (lax.broadcasted_iota(jnp.int32, (HIDDEN, N_HEADS * TILE), 0)
          // HEAD_DIM ==
          lax.broadcasted_iota(jnp.int32, (HIDDEN, N_HEADS * TILE), 1)
          // TILE).astype(BF)

    for l in range(N_LAYERS):
        mean = jnp.mean(xt * xt, axis=0, keepdims=True)
        xn = xt * lax.rsqrt(mean + EPS)

        qkvt = lax.dot_general(wqkv_ref[l], xn.astype(BF), dn(),
                               preferred_element_type=F32)
        qk = qkvt[:2 * HIDDEN]
        vt = qkvt[2 * HIDDEN:]
        qk = qk * cos_ref[...] + lax.dot_general(
            rot_ref[...], qk.astype(BF), dn(),
            preferred_element_type=F32) * sin_ref[...]
        qt = qk[:HIDDEN]
        kb = qk[HIDDEN:].astype(BF)
        vb = vt.astype(BF)

        attn_tiles = []
        for t in range(N_TILES):
            sl = slice(t * TILE, (t + 1) * TILE)
            qa = (qt[:, sl] * alives[t].astype(F32)).astype(BF)
            k4 = jnp.concatenate([kb[:, sl]] * N_HEADS, axis=1) * hm
            v4 = jnp.concatenate([vb[:, sl]] * N_HEADS, axis=1) * hm
            s = lax.dot_general(k4, qa, dt(),
                                preferred_element_type=F32)
            p = (jnp.exp(s) * pmask4s[t]).astype(BF)
            pv = lax.dot_general(v4, p, dn(),
                                 preferred_element_type=F32)
            den = lax.dot_general(hm, p, dn(),
                                  preferred_element_type=F32)
            attn_tiles.append(pv * pl.reciprocal(den, approx=True))
        attn = jnp.concatenate(attn_tiles, axis=1)

        xt = xt + lax.dot_general(wo_ref[l], attn.astype(BF), dn(),
                                  preferred_element_type=F32)

        mean2 = jnp.mean(xt * xt, axis=0, keepdims=True)
        xn2 = xt * lax.rsqrt(mean2 + EPS)
        gu = lax.dot_general(wgu_ref[l], xn2.astype(BF), dn(),
                             preferred_element_type=F32)
        g = gu[:INTER]
        u = gu[INTER:]
        hid = g * jax.nn.sigmoid(g) * u
        xt = xt + lax.dot_general(wd_ref[l], hid.astype(BF), dn(),
                                  preferred_element_type=F32)

    meanf = jnp.mean(xt * xt, axis=0, keepdims=True)
    xf = xt * lax.rsqrt(meanf + EPS)
    out_ref[...] = lax.dot_general(xf.astype(BF), wlm_ref[...], dt(),
                                   preferred_element_type=F32)


def kernel(input_ids, attention_mask, embed, final_norm, lm_head_T,
           ln1, wqkvT, woT, ln2, wguT, wdT, cos_qk, sin_qk, rot_qk):
    b = input_ids.shape[0]
    nblk = (b * SEQ) // ROWS
    nbseq = ROWS // SEQ

    scale = 1.0 / (HEAD_DIM ** 0.5)
    ln1c = jnp.swapaxes(ln1, 1, 2)
    ln2c = jnp.swapaxes(ln2, 1, 2)
    wq = wqkvT[:, :, :HIDDEN] * (ln1c * scale)
    wkv = wqkvT[:, :, HIDDEN:] * ln1c
    wqkv_t = jnp.swapaxes(jnp.concatenate([wq, wkv], axis=-1),
                          1, 2).astype(BF)
    wgu_t = jnp.swapaxes(wguT * ln2c, 1, 2).astype(BF)
    wo_t = jnp.swapaxes(woT, 1, 2).astype(BF)
    wd_t = jnp.swapaxes(wdT, 1, 2).astype(BF)
    wlm_f = (lm_head_T * jnp.swapaxes(final_norm, 0, 1)).astype(BF)
    emb_t = jnp.swapaxes(embed, 0, 1).astype(BF)
    rot_t = jnp.swapaxes(rot_qk, 0, 1).astype(BF)

    ids_rows = input_ids.reshape(nblk, 1, ROWS)
    mask_cols = attention_mask.reshape(nblk, ROWS, 1)
    cos_t = jnp.tile(jnp.swapaxes(cos_qk, 0, 1), (1, nbseq))
    sin_t = jnp.tile(jnp.swapaxes(sin_qk, 0, 1), (1, nbseq))

    shared = lambda shape: pl.BlockSpec(shape, lambda g: tuple(0 for _ in shape))

    out = pl.pallas_call(
        _fwd_kernel,
        out_shape=jax.ShapeDtypeStruct((b * SEQ, VOCAB), F32),
        grid=(nblk,),
        in_specs=[
            pl.BlockSpec((1, 1, ROWS), lambda g: (g, 0, 0)),
            pl.BlockSpec((1, ROWS, 1), lambda g: (g, 0, 0)),
            shared((HIDDEN, VOCAB)),
            shared((N_LAYERS, 3 * HIDDEN, HIDDEN)),
            shared((N_LAYERS, HIDDEN, HIDDEN)),
            shared((N_LAYERS, 2 * INTER, HIDDEN)),
            shared((N_LAYERS, HIDDEN, INTER)),
            shared((2 * HIDDEN, ROWS)),
            shared((2 * HIDDEN, ROWS)),
            shared((2 * HIDDEN, 2 * HIDDEN)),
            shared((HIDDEN, VOCAB)),
        ],
        out_specs=pl.BlockSpec((ROWS, VOCAB), lambda g: (g, 0)),
        compiler_params=pltpu.CompilerParams(
            dimension_semantics=("parallel",)),
    )(ids_rows, mask_cols, emb_t, wqkv_t, wo_t, wgu_t, wd_t,
      cos_t, sin_t, rot_t, wlm_f)

    return out.reshape(b, SEQ, VOCAB)


# speedup vs baseline: 96.9175x; 1.6206x over previous
import jax
import jax.numpy as jnp
from jax import lax
from jax.experimental import pallas as pl
from jax.experimental.pallas import tpu as pltpu

SEQ = 8
HIDDEN = 64
N_HEADS = 4
HEAD_DIM = HIDDEN // N_HEADS
INTER = 128
VOCAB = 256
N_LAYERS = 2
EPS = 1e-6
ROWS = 1024
TILE = 128
N_TILES = ROWS // TILE
BF = jnp.bfloat16
F32 = jnp.float32


def _fwd_kernel(ids_ref, mask_ref, embed_ref, wqkv_ref, wo_ref,
                wgu_ref, wd_ref, cos_ref, sin_ref, rot_ref,
                wlm_ref, out_ref):
    dn = lambda: (((1,), (0,)), ((), ()))
    dt = lambda: (((0,), (0,)), ((), ()))

    ids = ids_ref[0]
    vocab_iota = lax.broadcasted_iota(jnp.int32, (VOCAB, ROWS), 0)
    onehot = (vocab_iota == ids).astype(BF)
    xt = lax.dot_general(embed_ref[...], onehot, dn(),
                         preferred_element_type=F32)

    krow = lax.broadcasted_iota(jnp.int32, (TILE, TILE), 0)
    qcol = lax.broadcasted_iota(jnp.int32, (TILE, TILE), 1)
    samef = ((krow // SEQ) == (qcol // SEQ)).astype(F32)
    base = jnp.logical_and((krow // SEQ) == (qcol // SEQ), krow <= qcol)
    mkey = mask_ref[0]
    pmask4s, alives = [], []
    for t in range(N_TILES):
        mk = mkey[t * TILE:(t + 1) * TILE] > 0.5
        keepf = jnp.logical_and(base, mk).astype(F32)
        alive = jnp.max(keepf, axis=0, keepdims=True)
        pmask = jnp.maximum(keepf, samef * (1.0 - alive))
        pmask4s.append(jnp.concatenate([pmask] * N_HEADS, axis=0).astype(BF))
        alives.append(alive.astype(BF))

    hm = (lax.broadcasted_iota(jnp.int32, (HIDDEN, N_HEADS * TILE), 0)
          // HEAD_DIM ==
          lax.broadcasted_iota(jnp.int32, (HIDDEN, N_HEADS * TILE), 1)
          // TILE).astype(BF)

    for l in range(N_LAYERS):
        mean = jnp.mean(xt * xt, axis=0, keepdims=True)
        xn = xt * lax.rsqrt(mean + EPS)

        qkvt = lax.dot_general(wqkv_ref[l], xn.astype(BF), dn(),
                               preferred_element_type=F32)
        qk = qkvt[:2 * HIDDEN]
        vt = qkvt[2 * HIDDEN:]
        qk = qk * cos_ref[...] + lax.dot_general(
            rot_ref[...], qk.astype(BF), dn(),
            preferred_element_type=F32) * sin_ref[...]
        qt = qk[:HIDDEN]
        kb = qk[HIDDEN:].astype(BF)
        vb = vt.astype(BF)

        attn_tiles = []
        for t in range(N_TILES):
            sl = slice(t * TILE, (t + 1) * TILE)
            qa = (qt[:, sl] * alives[t].astype(F32)).astype(BF)
            k4 = jnp.concatenate([kb[:, sl]] * N_HEADS, axis=1) * hm
            v4 = jnp.concatenate([vb[:, sl]] * N_HEADS, axis=1) * hm
            s = lax.dot_general(k4, qa, dt(),
                                preferred_element_type=F32)
            p = (jnp.exp(s) * pmask4s[t]).astype(BF)
            pv = lax.dot_general(v4, p, dn(),
                                 preferred_element_type=F32)
            den = lax.dot_general(hm, p, dn(),
                                  preferred_element_type=F32)
            attn_tiles.append(pv * pl.reciprocal(den, approx=True))
        attn = jnp.concatenate(attn_tiles, axis=1)

        xt = xt + lax.dot_general(wo_ref[l], attn.astype(BF), dn(),
                                  preferred_element_type=F32)

        mean2 = jnp.mean(xt * xt, axis=0, keepdims=True)
        xn2 = xt * lax.rsqrt(mean2 + EPS)
        gu = lax.dot_general(wgu_ref[l], xn2.astype(BF), dn(),
                             preferred_element_type=F32)
        g = gu[:INTER]
        u = gu[INTER:]
        hid = g * jax.nn.sigmoid(g) * u
        xt = xt + lax.dot_general(wd_ref[l], hid.astype(BF), dn(),
                                  preferred_element_type=F32)

    meanf = jnp.mean(xt * xt, axis=0, keepdims=True)
    xf = xt * lax.rsqrt(meanf + EPS)
    out_ref[...] = lax.dot_general(xf.astype(BF), wlm_ref[...], dt(),
                                   preferred_element_type=F32)


def kernel(input_ids, attention_mask, embed, final_norm, lm_head_T,
           ln1, wqkvT, woT, ln2, wguT, wdT, cos_qk, sin_qk, rot_qk):
    b = input_ids.shape[0]
    nblk = (b * SEQ) // ROWS
    nbseq = ROWS // SEQ

    scale = 1.0 / (HEAD_DIM ** 0.5)
    ln1c = jnp.swapaxes(ln1, 1, 2)
    ln2c = jnp.swapaxes(ln2, 1, 2)
    wq = wqkvT[:, :, :HIDDEN] * (ln1c * scale)
    wkv = wqkvT[:, :, HIDDEN:] * ln1c
    wqkv_t = jnp.swapaxes(jnp.concatenate([wq, wkv], axis=-1),
                          1, 2).astype(BF)
    wgu_t = jnp.swapaxes(wguT * ln2c, 1, 2).astype(BF)
    wo_t = jnp.swapaxes(woT, 1, 2).astype(BF)
    wd_t = jnp.swapaxes(wdT, 1, 2).astype(BF)
    wlm_f = (lm_head_T * jnp.swapaxes(final_norm, 0, 1)).astype(BF)
    emb_t = jnp.swapaxes(embed, 0, 1).astype(BF)
    rot_t = jnp.swapaxes(rot_qk, 0, 1).astype(BF)

    ids_rows = input_ids.reshape(nblk, 1, ROWS)
    mask_cols = attention_mask.reshape(nblk, ROWS, 1)
    cos_t = jnp.tile(jnp.swapaxes(cos_qk, 0, 1), (1, nbseq))
    sin_t = jnp.tile(jnp.swapaxes(sin_qk, 0, 1), (1, nbseq))

    shared = lambda shape: pl.BlockSpec(shape, lambda g: tuple(0 for _ in shape))

    out = pl.pallas_call(
        _fwd_kernel,
        out_shape=jax.ShapeDtypeStruct((b * SEQ, VOCAB), F32),
        grid=(nblk,),
        in_specs=[
            pl.BlockSpec((1, 1, ROWS), lambda g: (g, 0, 0)),
            pl.BlockSpec((1, ROWS, 1), lambda g: (g, 0, 0)),
            shared((HIDDEN, VOCAB)),
            shared((N_LAYERS, 3 * HIDDEN, HIDDEN)),
            shared((N_LAYERS, HIDDEN, HIDDEN)),
            shared((N_LAYERS, 2 * INTER, HIDDEN)),
            shared((N_LAYERS, HIDDEN, INTER)),
            shared((2 * HIDDEN, ROWS)),
            shared((2 * HIDDEN, ROWS)),
            shared((2 * HIDDEN, 2 * HIDDEN)),
            shared((HIDDEN, VOCAB)),
        ],
        out_specs=pl.BlockSpec((ROWS, VOCAB), lambda g: (g, 0)),
        compiler_params=pltpu.CompilerParams(
            dimension_semantics=("parallel",)),
    )(ids_rows, mask_cols, emb_t, wqkv_t, wo_t, wgu_t, wd_t,
      cos_t, sin_t, rot_t, wlm_f)

    return out.reshape(b, SEQ, VOCAB)
